# baseline (speedup 1.0000x reference)
import functools

import jax
import jax.numpy as jnp
from jax import lax
from jax.experimental import pallas as pl
from jax.experimental.pallas import tpu as pltpu
from jax.experimental.pallas import tpu_sc as plsc

_N = 10000
_E = 320000
_H = 128
_G = 64
_C = 10

_NC = 2
_NS = 16
_NW = _NC * _NS
_LANES = 16
_EW = _E // _NW
_K = 128
_NCHW = 80
_EPAD = _NW * _NCHW * _K
_SUPCH = 40
_ROWS_T = 624
_ROWS_REM = _N - _NS * _ROWS_T


def _sc_agg(r, esrc, edst, ew, zeros):
    mesh = plsc.VectorSubcoreMesh(core_axis_name="c", subcore_axis_name="s",
                                  num_cores=_NC, num_subcores=_NS)

    @functools.partial(
        pl.kernel,
        out_type=jax.ShapeDtypeStruct((_NC * _N, _H), jnp.float32),
        mesh=mesh,
        scratch_types=[
            pltpu.VMEM((_SUPCH, _K), jnp.int32),
            pltpu.VMEM((_SUPCH, _K), jnp.int32),
            pltpu.VMEM((_SUPCH, _K), jnp.float32),
            pltpu.VMEM((_K, _H // 2), jnp.int32),
            pltpu.VMEM((_K, _H // 2), jnp.int32),
            pltpu.VMEM((_K, _H), jnp.float32),
            pltpu.VMEM_SHARED((_N, _H), jnp.float32),
            pltpu.SemaphoreType.DMA,
            pltpu.SemaphoreType.DMA,
        ],
        compiler_params=pltpu.CompilerParams(needs_layout_passes=False,
                                             use_tc_tiling_on_sc=False),
    )
    def k(r_hbm, s_hbm, d_hbm, w_hbm, z_hbm, out_hbm,
          src_v, dst_v, w_v, rows0_v, rows1_v, msg_v, acc, sem0, sem1):
        c = lax.axis_index("c")
        s = lax.axis_index("s")
        wid = c * _NS + s
        pltpu.sync_copy(z_hbm, acc.at[pl.ds(s * _ROWS_T, _ROWS_T)])

        @pl.when(s == _NS - 1)
        def _():
            pltpu.sync_copy(z_hbm.at[pl.ds(0, _ROWS_REM)],
                            acc.at[pl.ds(_NS * _ROWS_T, _ROWS_REM)])

        plsc.subcore_barrier()

        def start_gather(cl, buf, sem):
            return pltpu.async_copy(r_hbm.at[src_v.at[cl]], buf, sem)

        def wait_gather(cl, buf, sem):
            pltpu.make_async_copy(r_hbm.at[src_v.at[cl]], buf, sem).wait()

        def scale(cl, buf):
            def grp(g, carry):
                wg = w_v[cl, pl.ds(g * _LANES, _LANES)]
                for kk in range(_LANES):
                    wk = lax.gather(
                        wg, jnp.full((_LANES, 1), kk, jnp.int32),
                        lax.GatherDimensionNumbers(
                            offset_dims=(), collapsed_slice_dims=(0,),
                            start_index_map=(0,)),
                        (1,), mode=lax.GatherScatterMode.PROMISE_IN_BOUNDS)
                    e = g * _LANES + kk
                    for wb in range(_H // 2 // _LANES):
                        v = buf[e, pl.ds(wb * _LANES, _LANES)]
                        vb = plsc.bitcast(v, jnp.bfloat16)
                        va, vbb = plsc.unpack(vb,
                                              format=plsc.PackFormat.INTERLEAVED)
                        msg_v[e, pl.ds(wb * _LANES, _LANES)] = va * wk
                        msg_v[e, pl.ds(_H // 2 + wb * _LANES, _LANES)] = vbb * wk
                return carry

            lax.fori_loop(0, _K // _LANES, grp, 0)

        def process(cl, buf):
            scale(cl, buf)
            pltpu.sync_copy(msg_v, acc.at[dst_v.at[cl]], add=True)

        def sup_body(sp, carry):
            base_row = wid * _NCHW + sp * _SUPCH
            pltpu.sync_copy(s_hbm.at[pl.ds(base_row, _SUPCH)], src_v)
            pltpu.sync_copy(d_hbm.at[pl.ds(base_row, _SUPCH)], dst_v)
            pltpu.sync_copy(w_hbm.at[pl.ds(base_row, _SUPCH)], w_v)
            start_gather(0, rows0_v, sem0)

            def pair(i, carry2):
                c0 = 2 * i
                wait_gather(c0, rows0_v, sem0)
                start_gather(c0 + 1, rows1_v, sem1)
                process(c0, rows0_v)
                wait_gather(c0 + 1, rows1_v, sem1)

                @pl.when(c0 + 2 < _SUPCH)
                def _():
                    start_gather(c0 + 2, rows0_v, sem0)

                process(c0 + 1, rows1_v)
                return carry2

            lax.fori_loop(0, _SUPCH // 2, pair, 0)
            return carry

        lax.fori_loop(0, _NCHW // _SUPCH, sup_body, 0)

        plsc.subcore_barrier()
        pltpu.sync_copy(acc.at[pl.ds(s * _ROWS_T, _ROWS_T)],
                        out_hbm.at[pl.ds(c * _N + s * _ROWS_T, _ROWS_T)])

        @pl.when(s == _NS - 1)
        def _():
            pltpu.sync_copy(
                acc.at[pl.ds(_NS * _ROWS_T, _ROWS_REM)],
                out_hbm.at[pl.ds(c * _N + _NS * _ROWS_T, _ROWS_REM)])

    return k(r, esrc, edst, ew, zeros)


_NB = 10
_B = _N // _NB


def _pad_edges(src, dst, w):
    rows = _E // _K
    rows_pad = _NW * _NCHW

    def body(s_ref, d_ref, w_ref, so_ref, do_ref, wo_ref):
        spread = (lax.broadcasted_iota(jnp.int32, (rows_pad, _K), 0) * _K
                  + lax.broadcasted_iota(jnp.int32, (rows_pad, _K), 1)) % _N
        so_ref[...] = spread
        do_ref[...] = spread
        wo_ref[...] = jnp.zeros_like(wo_ref)
        so_ref[pl.ds(0, rows), :] = s_ref[...]
        do_ref[pl.ds(0, rows), :] = d_ref[...]
        wo_ref[pl.ds(0, rows), :] = w_ref[...]

    return pl.pallas_call(
        body,
        out_shape=[
            jax.ShapeDtypeStruct((rows_pad, _K), jnp.int32),
            jax.ShapeDtypeStruct((rows_pad, _K), jnp.int32),
            jax.ShapeDtypeStruct((rows_pad, _K), jnp.float32),
        ],
    )(src, dst, w)


def _pack_rows(v):
    vb = v.astype(jnp.bfloat16)
    lo = lax.bitcast_convert_type(vb[:, : _H // 2], jnp.uint16
                                  ).astype(jnp.uint32)
    hi = lax.bitcast_convert_type(vb[:, _H // 2:], jnp.uint16
                                  ).astype(jnp.uint32)
    return lax.bitcast_convert_type(lo | (hi << 16), jnp.int32)


def _relu_tc(x):
    def body(x_ref, o_ref):
        o_ref[...] = _pack_rows(jnp.maximum(x_ref[...], 0.0))

    return pl.pallas_call(
        body,
        grid=(_NB,),
        in_specs=[pl.BlockSpec((_B, _H), lambda j: (j, 0))],
        out_specs=pl.BlockSpec((_B, _H // 2), lambda j: (j, 0)),
        out_shape=jax.ShapeDtypeStruct((_N, _H // 2), jnp.int32),
    )(x)


def _dense_layer(h, parts, W1, b1, W2, b2, gamma, beta):
    def body(h_ref, p_ref, W1_ref, b1_ref, W2_ref, b2_ref, g_ref, be_ref,
             hout_ref, rout_ref, z2_scr, sums_scr):
        p = pl.program_id(0)
        j = pl.program_id(1)

        @pl.when(p == 0)
        def _():
            z = h_ref[...] + p_ref[0] + p_ref[1]
            z1 = jnp.maximum(
                lax.dot(z, W1_ref[...], preferred_element_type=jnp.float32)
                + b1_ref[...], 0.0)
            z2 = (lax.dot(z1, W2_ref[...], preferred_element_type=jnp.float32)
                  + b2_ref[...])
            z2_scr[pl.ds(j * _B, _B), :] = z2

            @pl.when(j == 0)
            def _():
                sums_scr[...] = jnp.zeros_like(sums_scr)

            sums_scr[0:1, :] += jnp.sum(z2, axis=0, keepdims=True)
            sums_scr[1:2, :] += jnp.sum(z2 * z2, axis=0, keepdims=True)

        @pl.when(p == 1)
        def _():
            z2 = z2_scr[pl.ds(j * _B, _B), :]
            mean = sums_scr[0:1, :] * (1.0 / _N)
            var = sums_scr[1:2, :] * (1.0 / _N) - mean * mean
            inv = lax.rsqrt(var + 1e-5)
            zn = (z2 - mean) * inv * g_ref[...] + be_ref[...]
            hn = h_ref[...] + jnp.maximum(zn, 0.0)
            hout_ref[...] = hn
            rout_ref[...] = _pack_rows(jnp.maximum(hn, 0.0))

    blk = lambda pp, j: (j, 0)
    full = lambda pp, j: (0, 0)
    return pl.pallas_call(
        body,
        grid=(2, _NB),
        in_specs=[
            pl.BlockSpec((_B, _H), blk),
            pl.BlockSpec((2, _B, _H), lambda pp, j: (0, j, 0)),
            pl.BlockSpec((_H, _H), full),
            pl.BlockSpec((1, _H), full),
            pl.BlockSpec((_H, _H), full),
            pl.BlockSpec((1, _H), full),
            pl.BlockSpec((1, _H), full),
            pl.BlockSpec((1, _H), full),
        ],
        out_specs=[
            pl.BlockSpec((_B, _H), blk),
            pl.BlockSpec((_B, _H // 2), blk),
        ],
        out_shape=[
            jax.ShapeDtypeStruct((_N, _H), jnp.float32),
            jax.ShapeDtypeStruct((_N, _H // 2), jnp.int32),
        ],
        scratch_shapes=[
            pltpu.VMEM((_N, _H), jnp.float32),
            pltpu.VMEM((2, _H), jnp.float32),
        ],
    )(h, parts, W1, b1, W2, b2, gamma, beta)


def _pool_head(h, batch2d, Wm1, bm1, Wm2p, bm2p):
    def body(h_ref, b_ref, W1_ref, b1_ref, W2_ref, b2_ref, o_ref, hsum, cnt):
        j = pl.program_id(0)

        @pl.when(j == 0)
        def _():
            hsum[...] = jnp.zeros_like(hsum)
            cnt[...] = jnp.zeros_like(cnt)

        oh = (b_ref[...] == lax.broadcasted_iota(jnp.int32, (1, _G), 1)
              ).astype(jnp.float32)
        hsum[...] += lax.dot_general(oh, h_ref[...], (((0,), (0,)), ((), ())),
                                     preferred_element_type=jnp.float32)
        cnt[...] += lax.dot_general(
            oh, jnp.ones((_B, 1), jnp.float32), (((0,), (0,)), ((), ())),
            preferred_element_type=jnp.float32)

        @pl.when(j == _NB - 1)
        def _():
            hg = hsum[...] / jnp.maximum(cnt[...], 1.0)
            t = jnp.maximum(
                lax.dot(hg, W1_ref[...], preferred_element_type=jnp.float32)
                + b1_ref[...], 0.0)
            o_ref[...] = (lax.dot(t, W2_ref[...],
                                  preferred_element_type=jnp.float32)
                          + b2_ref[...])

    full = lambda j: (0, 0)
    return pl.pallas_call(
        body,
        grid=(_NB,),
        in_specs=[
            pl.BlockSpec((_B, _H), lambda j: (j, 0)),
            pl.BlockSpec((_B, 1), lambda j: (j, 0)),
            pl.BlockSpec((_H, _H), full),
            pl.BlockSpec((1, _H), full),
            pl.BlockSpec((_H, _H), full),
            pl.BlockSpec((1, _H), full),
        ],
        out_specs=pl.BlockSpec((_G, _H), full),
        out_shape=jax.ShapeDtypeStruct((_G, _H), jnp.float32),
        scratch_shapes=[
            pltpu.VMEM((_G, _H), jnp.float32),
            pltpu.VMEM((_G, 1), jnp.float32),
        ],
    )(h, batch2d, Wm1, bm1, Wm2p, bm2p)


def kernel(x, edge_index, edge_weight, batch, params):
    src = edge_index[0].astype(jnp.int32).reshape(_E // _K, _K)
    dst = edge_index[1].astype(jnp.int32).reshape(_E // _K, _K)
    w2d = edge_weight.astype(jnp.float32).reshape(_E // _K, _K)
    esrc, edst, ew = _pad_edges(src, dst, w2d)
    zeros = jnp.zeros((_ROWS_T, _H), jnp.float32)
    batch2d = batch.astype(jnp.int32).reshape(_N, 1)

    h = x
    r = _relu_tc(x)
    for p in params["layers"]:
        part = _sc_agg(r, esrc, edst, ew, zeros).reshape(2, _N, _H)
        h, r = _dense_layer(
            h, part,
            p["W1"], p["b1"].reshape(1, _H),
            p["W2"], p["b2"].reshape(1, _H),
            p["gamma"].reshape(1, _H), p["beta"].reshape(1, _H))

    Wm1 = params["mlp"][0]["W"]
    bm1 = params["mlp"][0]["b"].reshape(1, _H)
    Wm2p = jnp.zeros((_H, _H), jnp.float32).at[:, :_C].set(params["mlp"][1]["W"])
    bm2p = jnp.zeros((1, _H), jnp.float32).at[0, :_C].set(params["mlp"][1]["b"])
    hg = _pool_head(h, batch2d, Wm1, bm1, Wm2p, bm2p)
    return hg[:, :_C]


# speedup vs baseline: 1.9259x; 1.9259x over previous
import functools

import jax
import jax.numpy as jnp
from jax import lax
from jax.experimental import pallas as pl
from jax.experimental.pallas import tpu as pltpu
from jax.experimental.pallas import tpu_sc as plsc

_N = 10000
_E = 320000
_H = 128
_G = 64
_C = 10

_NC = 2
_NS = 16
_NW = _NC * _NS
_LANES = 16
_EW = _E // _NW
_K = 128
_NCHW = 80
_EPAD = _NW * _NCHW * _K
_SUPCH = 40
_ROWS_T = 624
_ROWS_REM = _N - _NS * _ROWS_T


def _sc_agg(r, esrc, edst, ew, zeros):
    mesh = plsc.VectorSubcoreMesh(core_axis_name="c", subcore_axis_name="s",
                                  num_cores=_NC, num_subcores=_NS)

    @functools.partial(
        pl.kernel,
        out_type=jax.ShapeDtypeStruct((_NC * _N, _H), jnp.float32),
        mesh=mesh,
        scratch_types=[
            pltpu.VMEM((_SUPCH, _K), jnp.int32),
            pltpu.VMEM((_SUPCH, _K), jnp.int32),
            pltpu.VMEM((_SUPCH, _K), jnp.float32),
            pltpu.VMEM((_K, _H), jnp.float32),
            pltpu.VMEM((_K, _H), jnp.float32),
            pltpu.VMEM_SHARED((_N, _H), jnp.float32),
            pltpu.SemaphoreType.DMA,
            pltpu.SemaphoreType.DMA,
        ],
        compiler_params=pltpu.CompilerParams(needs_layout_passes=False),
    )
    def k(r_hbm, s_hbm, d_hbm, w_hbm, z_hbm, out_hbm,
          src_v, dst_v, w_v, rows0_v, rows1_v, acc, sem0, sem1):
        c = lax.axis_index("c")
        s = lax.axis_index("s")
        wid = c * _NS + s
        pltpu.sync_copy(z_hbm, acc.at[pl.ds(s * _ROWS_T, _ROWS_T)])

        @pl.when(s == _NS - 1)
        def _():
            pltpu.sync_copy(z_hbm.at[pl.ds(0, _ROWS_REM)],
                            acc.at[pl.ds(_NS * _ROWS_T, _ROWS_REM)])

        plsc.subcore_barrier()

        def start_gather(cl, buf, sem):
            return pltpu.async_copy(r_hbm.at[src_v.at[cl]], buf, sem)

        def wait_gather(cl, buf, sem):
            pltpu.make_async_copy(r_hbm.at[src_v.at[cl]], buf, sem).wait()

        def scale(cl, buf):
            def grp(g, carry):
                wg = w_v[cl, pl.ds(g * _LANES, _LANES)]
                for kk in range(_LANES):
                    wk = lax.gather(
                        wg, jnp.full((_LANES, 1), kk, jnp.int32),
                        lax.GatherDimensionNumbers(
                            offset_dims=(), collapsed_slice_dims=(0,),
                            start_index_map=(0,)),
                        (1,), mode=lax.GatherScatterMode.PROMISE_IN_BOUNDS)
                    e = g * _LANES + kk
                    for fb in range(_H // _LANES):
                        sl = pl.ds(fb * _LANES, _LANES)
                        buf[e, sl] = buf[e, sl] * wk
                return carry

            lax.fori_loop(0, _K // _LANES, grp, 0)

        def process(cl, buf):
            scale(cl, buf)
            pltpu.sync_copy(buf, acc.at[dst_v.at[cl]], add=True)

        def sup_body(sp, carry):
            base_row = wid * _NCHW + sp * _SUPCH
            pltpu.sync_copy(s_hbm.at[pl.ds(base_row, _SUPCH)], src_v)
            pltpu.sync_copy(d_hbm.at[pl.ds(base_row, _SUPCH)], dst_v)
            pltpu.sync_copy(w_hbm.at[pl.ds(base_row, _SUPCH)], w_v)
            start_gather(0, rows0_v, sem0)

            def pair(i, carry2):
                c0 = 2 * i
                wait_gather(c0, rows0_v, sem0)
                start_gather(c0 + 1, rows1_v, sem1)
                process(c0, rows0_v)
                wait_gather(c0 + 1, rows1_v, sem1)

                @pl.when(c0 + 2 < _SUPCH)
                def _():
                    start_gather(c0 + 2, rows0_v, sem0)

                process(c0 + 1, rows1_v)
                return carry2

            lax.fori_loop(0, _SUPCH // 2, pair, 0)
            return carry

        lax.fori_loop(0, _NCHW // _SUPCH, sup_body, 0)

        plsc.subcore_barrier()
        pltpu.sync_copy(acc.at[pl.ds(s * _ROWS_T, _ROWS_T)],
                        out_hbm.at[pl.ds(c * _N + s * _ROWS_T, _ROWS_T)])

        @pl.when(s == _NS - 1)
        def _():
            pltpu.sync_copy(
                acc.at[pl.ds(_NS * _ROWS_T, _ROWS_REM)],
                out_hbm.at[pl.ds(c * _N + _NS * _ROWS_T, _ROWS_REM)])

    return k(r, esrc, edst, ew, zeros)


_NB = 10
_B = _N // _NB


def _pad_edges(src, dst, w):
    rows = _E // _K
    rows_pad = _NW * _NCHW

    def body(s_ref, d_ref, w_ref, so_ref, do_ref, wo_ref):
        spread = (lax.broadcasted_iota(jnp.int32, (rows_pad, _K), 0) * _K
                  + lax.broadcasted_iota(jnp.int32, (rows_pad, _K), 1)) % _N
        so_ref[...] = spread
        do_ref[...] = spread
        wo_ref[...] = jnp.zeros_like(wo_ref)
        so_ref[pl.ds(0, rows), :] = s_ref[...]
        do_ref[pl.ds(0, rows), :] = d_ref[...]
        wo_ref[pl.ds(0, rows), :] = w_ref[...]

    return pl.pallas_call(
        body,
        out_shape=[
            jax.ShapeDtypeStruct((rows_pad, _K), jnp.int32),
            jax.ShapeDtypeStruct((rows_pad, _K), jnp.int32),
            jax.ShapeDtypeStruct((rows_pad, _K), jnp.float32),
        ],
    )(src, dst, w)


def _relu_tc(x):
    def body(x_ref, o_ref):
        o_ref[...] = jnp.maximum(x_ref[...], 0.0)

    return pl.pallas_call(
        body,
        grid=(_NB,),
        in_specs=[pl.BlockSpec((_B, _H), lambda j: (j, 0))],
        out_specs=pl.BlockSpec((_B, _H), lambda j: (j, 0)),
        out_shape=jax.ShapeDtypeStruct((_N, _H), jnp.float32),
    )(x)


def _dense_layer(h, parts, W1, b1, W2, b2, gamma, beta):
    def body(h_ref, p_ref, W1_ref, b1_ref, W2_ref, b2_ref, g_ref, be_ref,
             hout_ref, rout_ref, z2_scr, sums_scr):
        p = pl.program_id(0)
        j = pl.program_id(1)

        @pl.when(p == 0)
        def _():
            z = h_ref[...] + p_ref[0] + p_ref[1]
            z1 = jnp.maximum(
                lax.dot(z, W1_ref[...], preferred_element_type=jnp.float32)
                + b1_ref[...], 0.0)
            z2 = (lax.dot(z1, W2_ref[...], preferred_element_type=jnp.float32)
                  + b2_ref[...])
            z2_scr[pl.ds(j * _B, _B), :] = z2

            @pl.when(j == 0)
            def _():
                sums_scr[...] = jnp.zeros_like(sums_scr)

            sums_scr[0:1, :] += jnp.sum(z2, axis=0, keepdims=True)
            sums_scr[1:2, :] += jnp.sum(z2 * z2, axis=0, keepdims=True)

        @pl.when(p == 1)
        def _():
            z2 = z2_scr[pl.ds(j * _B, _B), :]
            mean = sums_scr[0:1, :] * (1.0 / _N)
            var = sums_scr[1:2, :] * (1.0 / _N) - mean * mean
            inv = lax.rsqrt(var + 1e-5)
            zn = (z2 - mean) * inv * g_ref[...] + be_ref[...]
            hn = h_ref[...] + jnp.maximum(zn, 0.0)
            hout_ref[...] = hn
            rout_ref[...] = jnp.maximum(hn, 0.0)

    blk = lambda pp, j: (j, 0)
    full = lambda pp, j: (0, 0)
    return pl.pallas_call(
        body,
        grid=(2, _NB),
        in_specs=[
            pl.BlockSpec((_B, _H), blk),
            pl.BlockSpec((2, _B, _H), lambda pp, j: (0, j, 0)),
            pl.BlockSpec((_H, _H), full),
            pl.BlockSpec((1, _H), full),
            pl.BlockSpec((_H, _H), full),
            pl.BlockSpec((1, _H), full),
            pl.BlockSpec((1, _H), full),
            pl.BlockSpec((1, _H), full),
        ],
        out_specs=[
            pl.BlockSpec((_B, _H), blk),
            pl.BlockSpec((_B, _H), blk),
        ],
        out_shape=[
            jax.ShapeDtypeStruct((_N, _H), jnp.float32),
            jax.ShapeDtypeStruct((_N, _H), jnp.float32),
        ],
        scratch_shapes=[
            pltpu.VMEM((_N, _H), jnp.float32),
            pltpu.VMEM((2, _H), jnp.float32),
        ],
    )(h, parts, W1, b1, W2, b2, gamma, beta)


def _pool_head(h, batch2d, Wm1, bm1, Wm2p, bm2p):
    def body(h_ref, b_ref, W1_ref, b1_ref, W2_ref, b2_ref, o_ref, hsum, cnt):
        j = pl.program_id(0)

        @pl.when(j == 0)
        def _():
            hsum[...] = jnp.zeros_like(hsum)
            cnt[...] = jnp.zeros_like(cnt)

        oh = (b_ref[...] == lax.broadcasted_iota(jnp.int32, (1, _G), 1)
              ).astype(jnp.float32)
        hsum[...] += lax.dot_general(oh, h_ref[...], (((0,), (0,)), ((), ())),
                                     preferred_element_type=jnp.float32)
        cnt[...] += lax.dot_general(
            oh, jnp.ones((_B, 1), jnp.float32), (((0,), (0,)), ((), ())),
            preferred_element_type=jnp.float32)

        @pl.when(j == _NB - 1)
        def _():
            hg = hsum[...] / jnp.maximum(cnt[...], 1.0)
            t = jnp.maximum(
                lax.dot(hg, W1_ref[...], preferred_element_type=jnp.float32)
                + b1_ref[...], 0.0)
            o_ref[...] = (lax.dot(t, W2_ref[...],
                                  preferred_element_type=jnp.float32)
                          + b2_ref[...])

    full = lambda j: (0, 0)
    return pl.pallas_call(
        body,
        grid=(_NB,),
        in_specs=[
            pl.BlockSpec((_B, _H), lambda j: (j, 0)),
            pl.BlockSpec((_B, 1), lambda j: (j, 0)),
            pl.BlockSpec((_H, _H), full),
            pl.BlockSpec((1, _H), full),
            pl.BlockSpec((_H, _H), full),
            pl.BlockSpec((1, _H), full),
        ],
        out_specs=pl.BlockSpec((_G, _H), full),
        out_shape=jax.ShapeDtypeStruct((_G, _H), jnp.float32),
        scratch_shapes=[
            pltpu.VMEM((_G, _H), jnp.float32),
            pltpu.VMEM((_G, 1), jnp.float32),
        ],
    )(h, batch2d, Wm1, bm1, Wm2p, bm2p)


def kernel(x, edge_index, edge_weight, batch, params):
    src = edge_index[0].astype(jnp.int32).reshape(_E // _K, _K)
    dst = edge_index[1].astype(jnp.int32).reshape(_E // _K, _K)
    w2d = edge_weight.astype(jnp.float32).reshape(_E // _K, _K)
    esrc, edst, ew = _pad_edges(src, dst, w2d)
    zeros = jnp.zeros((_ROWS_T, _H), jnp.float32)
    batch2d = batch.astype(jnp.int32).reshape(_N, 1)

    h = x
    r = _relu_tc(x)
    for p in params["layers"]:
        part = _sc_agg(r, esrc, edst, ew, zeros).reshape(2, _N, _H)
        h, r = _dense_layer(
            h, part,
            p["W1"], p["b1"].reshape(1, _H),
            p["W2"], p["b2"].reshape(1, _H),
            p["gamma"].reshape(1, _H), p["beta"].reshape(1, _H))

    Wm1 = params["mlp"][0]["W"]
    bm1 = params["mlp"][0]["b"].reshape(1, _H)
    Wm2p = jnp.zeros((_H, _H), jnp.float32).at[:, :_C].set(params["mlp"][1]["W"])
    bm2p = jnp.zeros((1, _H), jnp.float32).at[0, :_C].set(params["mlp"][1]["b"])
    hg = _pool_head(h, batch2d, Wm1, bm1, Wm2p, bm2p)
    return hg[:, :_C]


# speedup vs baseline: 1.9361x; 1.0053x over previous
import functools

import jax
import jax.numpy as jnp
from jax import lax
from jax.experimental import pallas as pl
from jax.experimental.pallas import tpu as pltpu
from jax.experimental.pallas import tpu_sc as plsc

_N = 10000
_E = 320000
_H = 128
_G = 64
_C = 10

_NC = 2
_NS = 16
_NW = _NC * _NS
_LANES = 16
_EW = _E // _NW
_K = 128
_NCHW = 80
_EPAD = _NW * _NCHW * _K
_SUPCH = 40
_ROWS_T = 624
_ROWS_REM = _N - _NS * _ROWS_T


def _sc_agg(r, esrc, edst, ew, zeros):
    mesh = plsc.VectorSubcoreMesh(core_axis_name="c", subcore_axis_name="s",
                                  num_cores=_NC, num_subcores=_NS)

    @functools.partial(
        pl.kernel,
        out_type=jax.ShapeDtypeStruct((_NC * _N, _H), jnp.float32),
        mesh=mesh,
        scratch_types=[
            pltpu.VMEM((_SUPCH, _K), jnp.int32),
            pltpu.VMEM((_SUPCH, _K), jnp.int32),
            pltpu.VMEM((_SUPCH, _K), jnp.float32),
            pltpu.VMEM((_K, _H), jnp.float32),
            pltpu.VMEM((_K, _H), jnp.float32),
            pltpu.VMEM_SHARED((_N, _H), jnp.float32),
            pltpu.SemaphoreType.DMA,
            pltpu.SemaphoreType.DMA,
        ],
        compiler_params=pltpu.CompilerParams(needs_layout_passes=False),
    )
    def k(r_hbm, s_hbm, d_hbm, w_hbm, z_hbm, out_hbm,
          src_v, dst_v, w_v, rows0_v, rows1_v, acc, sem0, sem1):
        c = lax.axis_index("c")
        s = lax.axis_index("s")
        wid = c * _NS + s
        pltpu.sync_copy(z_hbm, acc.at[pl.ds(s * _ROWS_T, _ROWS_T)])

        @pl.when(s == _NS - 1)
        def _():
            pltpu.sync_copy(z_hbm.at[pl.ds(0, _ROWS_REM)],
                            acc.at[pl.ds(_NS * _ROWS_T, _ROWS_REM)])

        plsc.subcore_barrier()

        def start_gather(cl, buf, sem):
            return pltpu.async_copy(r_hbm.at[src_v.at[cl]], buf, sem)

        def wait_gather(cl, buf, sem):
            pltpu.make_async_copy(r_hbm.at[src_v.at[cl]], buf, sem).wait()

        def scale(cl, buf):
            def grp(g, carry):
                wg = w_v[cl, pl.ds(g * _LANES, _LANES)]
                for kk in range(_LANES):
                    wk = lax.gather(
                        wg, jnp.full((_LANES, 1), kk, jnp.int32),
                        lax.GatherDimensionNumbers(
                            offset_dims=(), collapsed_slice_dims=(0,),
                            start_index_map=(0,)),
                        (1,), mode=lax.GatherScatterMode.PROMISE_IN_BOUNDS)
                    e = g * _LANES + kk
                    for fb in range(_H // _LANES):
                        sl = pl.ds(fb * _LANES, _LANES)
                        buf[e, sl] = buf[e, sl] * wk
                return carry

            lax.fori_loop(0, _K // _LANES, grp, 0)

        def process(cl, buf):
            scale(cl, buf)
            pltpu.sync_copy(buf, acc.at[dst_v.at[cl]], add=True)

        def sup_body(sp, carry):
            base_row = wid * _NCHW + sp * _SUPCH
            pltpu.sync_copy(s_hbm.at[pl.ds(base_row, _SUPCH)], src_v)
            pltpu.sync_copy(d_hbm.at[pl.ds(base_row, _SUPCH)], dst_v)
            pltpu.sync_copy(w_hbm.at[pl.ds(base_row, _SUPCH)], w_v)
            start_gather(0, rows0_v, sem0)

            def pair(i, carry2):
                c0 = 2 * i
                wait_gather(c0, rows0_v, sem0)
                start_gather(c0 + 1, rows1_v, sem1)
                process(c0, rows0_v)
                wait_gather(c0 + 1, rows1_v, sem1)

                @pl.when(c0 + 2 < _SUPCH)
                def _():
                    start_gather(c0 + 2, rows0_v, sem0)

                process(c0 + 1, rows1_v)
                return carry2

            lax.fori_loop(0, _SUPCH // 2, pair, 0)
            return carry

        lax.fori_loop(0, _NCHW // _SUPCH, sup_body, 0)

        plsc.subcore_barrier()
        pltpu.sync_copy(acc.at[pl.ds(s * _ROWS_T, _ROWS_T)],
                        out_hbm.at[pl.ds(c * _N + s * _ROWS_T, _ROWS_T)])

        @pl.when(s == _NS - 1)
        def _():
            pltpu.sync_copy(
                acc.at[pl.ds(_NS * _ROWS_T, _ROWS_REM)],
                out_hbm.at[pl.ds(c * _N + _NS * _ROWS_T, _ROWS_REM)])

    return k(r, esrc, edst, ew, zeros)


_NB = 10
_B = _N // _NB


def _pad_edges(src, dst, w):
    rows = _E // _K
    rows_pad = _NW * _NCHW

    def body(s_ref, d_ref, w_ref, so_ref, do_ref, wo_ref):
        spread = (lax.broadcasted_iota(jnp.int32, (rows_pad, _K), 0) * _K
                  + lax.broadcasted_iota(jnp.int32, (rows_pad, _K), 1)) % _N
        so_ref[...] = spread
        do_ref[...] = spread
        wo_ref[...] = jnp.zeros_like(wo_ref)
        so_ref[pl.ds(0, rows), :] = s_ref[...]
        do_ref[pl.ds(0, rows), :] = d_ref[...]
        wo_ref[pl.ds(0, rows), :] = w_ref[...]

    return pl.pallas_call(
        body,
        out_shape=[
            jax.ShapeDtypeStruct((rows_pad, _K), jnp.int32),
            jax.ShapeDtypeStruct((rows_pad, _K), jnp.int32),
            jax.ShapeDtypeStruct((rows_pad, _K), jnp.float32),
        ],
    )(src, dst, w)


def _relu_tc(x):
    def body(x_ref, o_ref):
        o_ref[...] = jnp.maximum(x_ref[...], 0.0)

    return pl.pallas_call(
        body,
        grid=(_NB,),
        in_specs=[pl.BlockSpec((_B, _H), lambda j: (j, 0))],
        out_specs=pl.BlockSpec((_B, _H), lambda j: (j, 0)),
        out_shape=jax.ShapeDtypeStruct((_N, _H), jnp.float32),
    )(x)


def _dense_layer(h, parts, W1, b1, W2, b2, gamma, beta):
    def body(h_ref, p_ref, W1_ref, b1_ref, W2_ref, b2_ref, g_ref, be_ref,
             hout_ref, rout_ref, z2_scr, sums_scr):
        p = pl.program_id(0)
        j = pl.program_id(1)

        @pl.when(p == 0)
        def _():
            z = h_ref[...] + p_ref[0] + p_ref[1]
            z1 = jnp.maximum(
                lax.dot(z.astype(jnp.bfloat16),
                        W1_ref[...].astype(jnp.bfloat16),
                        preferred_element_type=jnp.float32)
                + b1_ref[...], 0.0)
            z2 = (lax.dot(z1.astype(jnp.bfloat16),
                          W2_ref[...].astype(jnp.bfloat16),
                          preferred_element_type=jnp.float32)
                  + b2_ref[...])
            z2_scr[pl.ds(j * _B, _B), :] = z2

            @pl.when(j == 0)
            def _():
                sums_scr[...] = jnp.zeros_like(sums_scr)

            sums_scr[0:1, :] += jnp.sum(z2, axis=0, keepdims=True)
            sums_scr[1:2, :] += jnp.sum(z2 * z2, axis=0, keepdims=True)

        @pl.when(p == 1)
        def _():
            z2 = z2_scr[pl.ds(j * _B, _B), :]
            mean = sums_scr[0:1, :] * (1.0 / _N)
            var = sums_scr[1:2, :] * (1.0 / _N) - mean * mean
            inv = lax.rsqrt(var + 1e-5)
            zn = (z2 - mean) * inv * g_ref[...] + be_ref[...]
            hn = h_ref[...] + jnp.maximum(zn, 0.0)
            hout_ref[...] = hn
            rout_ref[...] = jnp.maximum(hn, 0.0)

    blk = lambda pp, j: (j, 0)
    full = lambda pp, j: (0, 0)
    return pl.pallas_call(
        body,
        grid=(2, _NB),
        in_specs=[
            pl.BlockSpec((_B, _H), blk),
            pl.BlockSpec((2, _B, _H), lambda pp, j: (0, j, 0)),
            pl.BlockSpec((_H, _H), full),
            pl.BlockSpec((1, _H), full),
            pl.BlockSpec((_H, _H), full),
            pl.BlockSpec((1, _H), full),
            pl.BlockSpec((1, _H), full),
            pl.BlockSpec((1, _H), full),
        ],
        out_specs=[
            pl.BlockSpec((_B, _H), blk),
            pl.BlockSpec((_B, _H), blk),
        ],
        out_shape=[
            jax.ShapeDtypeStruct((_N, _H), jnp.float32),
            jax.ShapeDtypeStruct((_N, _H), jnp.float32),
        ],
        scratch_shapes=[
            pltpu.VMEM((_N, _H), jnp.float32),
            pltpu.VMEM((2, _H), jnp.float32),
        ],
    )(h, parts, W1, b1, W2, b2, gamma, beta)


def _pool_head(h, batch2d, Wm1, bm1, Wm2p, bm2p):
    def body(h_ref, b_ref, W1_ref, b1_ref, W2_ref, b2_ref, o_ref, hsum, cnt):
        j = pl.program_id(0)

        @pl.when(j == 0)
        def _():
            hsum[...] = jnp.zeros_like(hsum)
            cnt[...] = jnp.zeros_like(cnt)

        oh = (b_ref[...] == lax.broadcasted_iota(jnp.int32, (1, _G), 1)
              ).astype(jnp.float32)
        hsum[...] += lax.dot_general(oh, h_ref[...], (((0,), (0,)), ((), ())),
                                     preferred_element_type=jnp.float32)
        cnt[...] += lax.dot_general(
            oh, jnp.ones((_B, 1), jnp.float32), (((0,), (0,)), ((), ())),
            preferred_element_type=jnp.float32)

        @pl.when(j == _NB - 1)
        def _():
            hg = hsum[...] / jnp.maximum(cnt[...], 1.0)
            t = jnp.maximum(
                lax.dot(hg, W1_ref[...], preferred_element_type=jnp.float32)
                + b1_ref[...], 0.0)
            o_ref[...] = (lax.dot(t, W2_ref[...],
                                  preferred_element_type=jnp.float32)
                          + b2_ref[...])

    full = lambda j: (0, 0)
    return pl.pallas_call(
        body,
        grid=(_NB,),
        in_specs=[
            pl.BlockSpec((_B, _H), lambda j: (j, 0)),
            pl.BlockSpec((_B, 1), lambda j: (j, 0)),
            pl.BlockSpec((_H, _H), full),
            pl.BlockSpec((1, _H), full),
            pl.BlockSpec((_H, _H), full),
            pl.BlockSpec((1, _H), full),
        ],
        out_specs=pl.BlockSpec((_G, _H), full),
        out_shape=jax.ShapeDtypeStruct((_G, _H), jnp.float32),
        scratch_shapes=[
            pltpu.VMEM((_G, _H), jnp.float32),
            pltpu.VMEM((_G, 1), jnp.float32),
        ],
    )(h, batch2d, Wm1, bm1, Wm2p, bm2p)


def kernel(x, edge_index, edge_weight, batch, params):
    src = edge_index[0].astype(jnp.int32).reshape(_E // _K, _K)
    dst = edge_index[1].astype(jnp.int32).reshape(_E // _K, _K)
    w2d = edge_weight.astype(jnp.float32).reshape(_E // _K, _K)
    esrc, edst, ew = _pad_edges(src, dst, w2d)
    zeros = jnp.zeros((_ROWS_T, _H), jnp.float32)
    batch2d = batch.astype(jnp.int32).reshape(_N, 1)

    h = x
    r = _relu_tc(x)
    for p in params["layers"]:
        part = _sc_agg(r, esrc, edst, ew, zeros).reshape(2, _N, _H)
        h, r = _dense_layer(
            h, part,
            p["W1"], p["b1"].reshape(1, _H),
            p["W2"], p["b2"].reshape(1, _H),
            p["gamma"].reshape(1, _H), p["beta"].reshape(1, _H))

    Wm1 = params["mlp"][0]["W"]
    bm1 = params["mlp"][0]["b"].reshape(1, _H)
    Wm2p = jnp.zeros((_H, _H), jnp.float32).at[:, :_C].set(params["mlp"][1]["W"])
    bm2p = jnp.zeros((1, _H), jnp.float32).at[0, :_C].set(params["mlp"][1]["b"])
    hg = _pool_head(h, batch2d, Wm1, bm1, Wm2p, bm2p)
    return hg[:, :_C]
